# Initial kernel scaffold; baseline (speedup 1.0000x reference)
import functools

import jax
import jax.numpy as jnp
from jax import lax
from jax.experimental import pallas as pl
from jax.experimental.pallas import tpu as pltpu
from jax.experimental.pallas import tpu_sc as plsc

L = 16
NUM_WORKERS = 32
CHUNK_B = 64
IDX_PER_STREAM = 128


def _cp_kernel(H, V, R, B, coords_hbm, offs_hbm, factors_hbm, weights_hbm,
               bias_hbm, out_hbm, cvm, offs_vm, fidx_vm, rows_vm, wvm, bvm,
               out_vm, sem):
    per_tile = B // NUM_WORKERS
    n_chunks = per_tile // CHUNK_B
    idx_per_chunk = CHUNK_B * H
    n_streams = idx_per_chunk // IDX_PER_STREAM
    n_groups = CHUNK_B // L

    wid = lax.axis_index("s") * 2 + lax.axis_index("c")

    pltpu.sync_copy(offs_hbm, offs_vm)
    pltpu.sync_copy(weights_hbm, wvm)
    pltpu.sync_copy(bias_hbm, bvm)

    iota = lax.iota(jnp.int32, L)
    bias_vec = bvm[...]

    def chunk_body(c):
        base = (wid * per_tile + c * CHUNK_B) * H
        pltpu.sync_copy(coords_hbm.at[pl.ds(base, idx_per_chunk)], cvm)
        for i in range(idx_per_chunk // L):
            j, col = (i * L) // IDX_PER_STREAM, (i * L) % IDX_PER_STREAM
            sl = pl.ds(i * L, L)
            fidx_vm[j, pl.ds(col, L)] = cvm[sl] + offs_vm[sl]
        copies = []
        for j in range(n_streams):
            copies.append(
                pltpu.async_copy(
                    factors_hbm.at[fidx_vm.at[j]],
                    rows_vm.at[pl.ds(j * IDX_PER_STREAM, IDX_PER_STREAM)],
                    sem,
                ))
        for cp in copies:
            cp.wait()

        def group_body(g):
            row0 = (g * L + iota) * H
            out_vec = bias_vec
            for r in range(R):
                cols = (iota + r) & (R - 1)
                acc = plsc.load_gather(rows_vm, [row0, cols])
                for m in range(1, H):
                    acc = acc * plsc.load_gather(rows_vm, [row0 + m, cols])
                out_vec = out_vec + acc * plsc.load_gather(wvm, [cols])
            out_vm[pl.ds(c * CHUNK_B + g * L, L)] = out_vec

        pl.loop(0, n_groups)(group_body)

    pl.loop(0, n_chunks)(chunk_body)
    pltpu.sync_copy(out_vm, out_hbm.at[pl.ds(wid * per_tile, per_tile)])


@jax.jit
def kernel(coords, factors, weights, bias):
    B, H = coords.shape
    _, V, R = factors.shape
    assert R == 32 and B % NUM_WORKERS == 0
    per_tile = B // NUM_WORKERS

    coords_flat = coords.reshape(B * H)
    factors_flat = factors.reshape(H * V, R)
    offs = jnp.tile(jnp.arange(H, dtype=jnp.int32) * V, CHUNK_B)
    bias16 = jnp.broadcast_to(bias.astype(jnp.float32), (L,))

    mesh = plsc.VectorSubcoreMesh(core_axis_name="c", subcore_axis_name="s")
    run = pl.kernel(
        functools.partial(_cp_kernel, H, V, R, B),
        out_type=jax.ShapeDtypeStruct((B,), jnp.float32),
        mesh=mesh,
        scratch_types=[
            pltpu.VMEM((CHUNK_B * H,), jnp.int32),
            pltpu.VMEM((CHUNK_B * H,), jnp.int32),
            pltpu.VMEM((CHUNK_B * H // IDX_PER_STREAM, IDX_PER_STREAM),
                       jnp.int32),
            pltpu.VMEM((CHUNK_B * H, R), jnp.float32),
            pltpu.VMEM((R,), jnp.float32),
            pltpu.VMEM((L,), jnp.float32),
            pltpu.VMEM((per_tile,), jnp.float32),
            pltpu.SemaphoreType.DMA,
        ],
    )
    return run(coords_flat, offs, factors_flat, weights, bias16)


# speedup vs baseline: 1.1019x; 1.1019x over previous
import functools

import jax
import jax.numpy as jnp
from jax import lax
from jax.experimental import pallas as pl
from jax.experimental.pallas import tpu as pltpu
from jax.experimental.pallas import tpu_sc as plsc

L = 16
NUM_WORKERS = 32
CHUNK_B = 64
IDX_PER_STREAM = 128


def _cp_kernel(H, V, R, B, coords_hbm, offs_hbm, factors_hbm, weights_hbm,
               bias_hbm, out_hbm, cvm, offs_vm, fidx_vm, rows_vm, wvm, bvm,
               pbuf_vm, out_vm, sem):
    per_tile = B // NUM_WORKERS
    n_chunks = per_tile // CHUNK_B
    idx_per_chunk = CHUNK_B * H
    n_streams = idx_per_chunk // IDX_PER_STREAM
    n_groups = CHUNK_B // L

    wid = lax.axis_index("s") * 2 + lax.axis_index("c")

    pltpu.sync_copy(offs_hbm, offs_vm)
    pltpu.sync_copy(weights_hbm, wvm)
    pltpu.sync_copy(bias_hbm, bvm)

    iota = lax.iota(jnp.int32, L)
    iota17 = iota * (L + 1)
    bias_vec = bvm[...]
    w0 = wvm[pl.ds(0, L)]
    w1 = wvm[pl.ds(L, L)]

    def chunk_body(c):
        base = (wid * per_tile + c * CHUNK_B) * H
        pltpu.sync_copy(coords_hbm.at[pl.ds(base, idx_per_chunk)], cvm)
        for i in range(idx_per_chunk // L):
            j, col = (i * L) // IDX_PER_STREAM, (i * L) % IDX_PER_STREAM
            sl = pl.ds(i * L, L)
            fidx_vm[j, pl.ds(col, L)] = cvm[sl] + offs_vm[sl]
        copies = []
        for j in range(n_streams):
            copies.append(
                pltpu.async_copy(
                    factors_hbm.at[fidx_vm.at[j]],
                    rows_vm.at[pl.ds(j * IDX_PER_STREAM, IDX_PER_STREAM)],
                    sem,
                ))
        for cp in copies:
            cp.wait()

        def group_body(g):
            for b in range(L):
                row = (g * L + b) * H
                acc0 = rows_vm[row, pl.ds(0, L)]
                acc1 = rows_vm[row, pl.ds(L, L)]
                for m in range(1, H):
                    acc0 = acc0 * rows_vm[row + m, pl.ds(0, L)]
                    acc1 = acc1 * rows_vm[row + m, pl.ds(L, L)]
                pbuf_vm[pl.ds(b * (L + 1), L)] = acc0 * w0 + acc1 * w1
            out_vec = bias_vec
            for l in range(L):
                out_vec = out_vec + plsc.load_gather(pbuf_vm, [iota17 + l])
            out_vm[pl.ds(c * CHUNK_B + g * L, L)] = out_vec

        pl.loop(0, n_groups)(group_body)

    pl.loop(0, n_chunks)(chunk_body)
    pltpu.sync_copy(out_vm, out_hbm.at[pl.ds(wid * per_tile, per_tile)])


@jax.jit
def kernel(coords, factors, weights, bias):
    B, H = coords.shape
    _, V, R = factors.shape
    assert R == 32 and B % NUM_WORKERS == 0
    per_tile = B // NUM_WORKERS

    coords_flat = coords.reshape(B * H)
    factors_flat = factors.reshape(H * V, R)
    offs = jnp.tile(jnp.arange(H, dtype=jnp.int32) * V, CHUNK_B)
    bias16 = jnp.broadcast_to(bias.astype(jnp.float32), (L,))

    mesh = plsc.VectorSubcoreMesh(core_axis_name="c", subcore_axis_name="s",
                                  num_cores=2, num_subcores=16)
    run = pl.kernel(
        functools.partial(_cp_kernel, H, V, R, B),
        out_type=jax.ShapeDtypeStruct((B,), jnp.float32),
        mesh=mesh,
        compiler_params=pltpu.CompilerParams(needs_layout_passes=False,
                                             use_tc_tiling_on_sc=False),
        scratch_types=[
            pltpu.VMEM((CHUNK_B * H,), jnp.int32),
            pltpu.VMEM((CHUNK_B * H,), jnp.int32),
            pltpu.VMEM((CHUNK_B * H // IDX_PER_STREAM, IDX_PER_STREAM),
                       jnp.int32),
            pltpu.VMEM((CHUNK_B * H, R), jnp.float32),
            pltpu.VMEM((R,), jnp.float32),
            pltpu.VMEM((L,), jnp.float32),
            pltpu.VMEM((L * (L + 1),), jnp.float32),
            pltpu.VMEM((per_tile,), jnp.float32),
            pltpu.SemaphoreType.DMA,
        ],
    )
    return run(coords_flat, offs, factors_flat, weights, bias16)
